# baseline (speedup 1.0000x reference)
import jax
import jax.numpy as jnp
from jax import lax
from jax.experimental import pallas as pl
from jax.experimental.pallas import tpu as pltpu
from jax.experimental.pallas import tpu_sc as plsc

N = 100000
NC = 2
NS = 16
NW = NC * NS
LANES = 16

CHUNK = 3136
TAIL = N - (NW - 1) * CHUNK
assert TAIL > 0 and TAIL % LANES == 0 and CHUNK % LANES == 0


def _body(x_hbm, s_hbm, b_hbm, out_hbm, xv, sv, bv):
    wid = lax.axis_index("s") * NC + lax.axis_index("c")
    pltpu.sync_copy(s_hbm, sv)
    pltpu.sync_copy(b_hbm, bv)
    s = sv[...]
    b = bv[...]

    def run(base, size):
        pltpu.sync_copy(x_hbm.at[pl.ds(base, size)], xv.at[pl.ds(0, size)])

        def step(j, carry):
            sl = pl.ds(j * LANES, LANES)
            xv[sl] = s * xv[sl] + b
            return carry

        lax.fori_loop(0, size // LANES, step, 0)
        pltpu.sync_copy(xv.at[pl.ds(0, size)], out_hbm.at[pl.ds(base, size)])

    @pl.when(wid < NW - 1)
    def _():
        run(wid * CHUNK, CHUNK)

    @pl.when(wid == NW - 1)
    def _():
        run((NW - 1) * CHUNK, TAIL)


@jax.jit
def _scale_shift(x, s16, b16):
    kern = pl.kernel(
        _body,
        out_type=jax.ShapeDtypeStruct((N,), jnp.float32),
        mesh=plsc.VectorSubcoreMesh(core_axis_name="c", subcore_axis_name="s"),
        scratch_types=[
            pltpu.VMEM((CHUNK,), jnp.float32),
            pltpu.VMEM((LANES,), jnp.float32),
            pltpu.VMEM((LANES,), jnp.float32),
        ],
    )
    return kern(x, s16, b16)


def kernel(x, head, scale, shift):
    s16 = jnp.broadcast_to(jnp.reshape(scale, (1,)), (LANES,))
    b16 = jnp.broadcast_to(jnp.reshape(shift, (1,)), (LANES,))
    return _scale_shift(x, s16, b16)


# speedup vs baseline: 3.5416x; 3.5416x over previous
import jax
import jax.numpy as jnp
from jax.experimental import pallas as pl
from jax.experimental.pallas import tpu as pltpu

N = 100000
BLOCK = 12288


def _body(s_ref, b_ref, x_ref, o_ref):
    o_ref[...] = x_ref[...] * s_ref[0] + b_ref[0]


@jax.jit
def _scale_shift(x, s1, b1):
    grid = (N + BLOCK - 1) // BLOCK
    return pl.pallas_call(
        _body,
        out_shape=jax.ShapeDtypeStruct((N,), jnp.float32),
        grid=(grid,),
        in_specs=[
            pl.BlockSpec(memory_space=pltpu.SMEM),
            pl.BlockSpec(memory_space=pltpu.SMEM),
            pl.BlockSpec((BLOCK,), lambda i: (i,)),
        ],
        out_specs=pl.BlockSpec((BLOCK,), lambda i: (i,)),
    )(s1, b1, x)


def kernel(x, head, scale, shift):
    s1 = jnp.reshape(scale, (1,))
    b1 = jnp.reshape(shift, (1,))
    return _scale_shift(x, s1, b1)


# speedup vs baseline: 7.4741x; 2.1104x over previous
import jax
import jax.numpy as jnp
from jax.experimental import pallas as pl
from jax.experimental.pallas import tpu as pltpu

N = 100000
BLOCK = 12288


def _body(s_ref, b_ref, x_ref, o_ref):
    o_ref[...] = x_ref[...] * s_ref[0] + b_ref[0]


@jax.jit
def _scale_shift(x, s1, b1):
    return pl.pallas_call(
        _body,
        out_shape=jax.ShapeDtypeStruct((N,), jnp.float32),
        in_specs=[
            pl.BlockSpec(memory_space=pltpu.SMEM),
            pl.BlockSpec(memory_space=pltpu.SMEM),
            pl.BlockSpec((N,), lambda: (0,)),
        ],
        out_specs=pl.BlockSpec((N,), lambda: (0,)),
    )(s1, b1, x)


def kernel(x, head, scale, shift):
    s1 = jnp.reshape(scale, (1,))
    b1 = jnp.reshape(shift, (1,))
    return _scale_shift(x, s1, b1)
